# Initial kernel scaffold; baseline (speedup 1.0000x reference)
import jax
import jax.numpy as jnp
from jax.experimental import pallas as pl


def kernel(x_btd, layer0_w_ih_t, layer0_bias_flat, layer0_whh_g, layer1_wih_g, layer1_whh_g, layer1_bias_g, layer2_wih_g, layer2_whh_g, layer2_bias_g, fc_w_t, fc_b):
    raise NotImplementedError("write your pallas kernel here")


# speedup vs baseline: 1.0243x; 1.0243x over previous
import jax
import jax.numpy as jnp
from jax import lax
from jax.experimental import pallas as pl
from jax.experimental.pallas import tpu as pltpu


def _make_body(T, BB, D, H, O, unroll):
    G = 4 * H

    def body(x_ref, w0_ref, b0_ref, whh0_ref, wih1_ref, whh1_ref, b1_ref,
             wih2_ref, whh2_ref, b2_ref, fcw_ref, fcb_ref, out_ref,
             xg_scr, h_all_scr):
        x2d = x_ref[...].reshape(T * BB, D).astype(jnp.bfloat16)
        xg_scr[...] = (jnp.dot(x2d, w0_ref[...],
                               preferred_element_type=jnp.float32)
                       + b0_ref[...])

        def run_layer(whh_ref, store_h):
            whh = whh_ref[...]

            def step(t, carry):
                h, c = carry
                gates = (xg_scr[pl.ds(t * BB, BB), :]
                         + jnp.dot(h, whh, preferred_element_type=jnp.float32))
                i = jax.nn.sigmoid(gates[:, 0:H])
                f = jax.nn.sigmoid(gates[:, H:2 * H])
                g = jnp.tanh(gates[:, 2 * H:3 * H])
                o = jax.nn.sigmoid(gates[:, 3 * H:4 * H])
                c_new = f * c + i * g
                h_new = (o * jnp.tanh(c_new)).astype(jnp.bfloat16)
                if store_h:
                    h_all_scr[pl.ds(t * BB, BB), :] = h_new
                return h_new, c_new

            z = (jnp.zeros((BB, H), jnp.bfloat16),
                 jnp.zeros((BB, H), jnp.float32))
            return lax.fori_loop(0, T, step, z, unroll=unroll)

        run_layer(whh0_ref, True)

        xg_scr[...] = (jnp.dot(h_all_scr[...], wih1_ref[...],
                               preferred_element_type=jnp.float32)
                       + b1_ref[...])
        run_layer(whh1_ref, True)

        xg_scr[...] = (jnp.dot(h_all_scr[...], wih2_ref[...],
                               preferred_element_type=jnp.float32)
                       + b2_ref[...])
        h_last, _ = run_layer(whh2_ref, False)

        out_ref[...] = (jnp.dot(h_last, fcw_ref[...],
                                preferred_element_type=jnp.float32)
                        + fcb_ref[...])

    return body


def _gate_major(w_g):
    g, hin, hout = w_g.shape
    return jnp.transpose(w_g, (1, 0, 2)).reshape(hin, g * hout)


def kernel(x_btd, layer0_w_ih_t, layer0_bias_flat, layer0_whh_g,
           layer1_wih_g, layer1_whh_g, layer1_bias_g,
           layer2_wih_g, layer2_whh_g, layer2_bias_g,
           fc_w_t, fc_b):
    B, T, D = x_btd.shape
    H = layer0_whh_g.shape[-1]
    O = fc_w_t.shape[1]
    G = 4 * H
    NCORES = 2
    BB = B // NCORES

    x = jnp.transpose(x_btd, (1, 0, 2))
    whh0 = _gate_major(layer0_whh_g)
    wih1 = _gate_major(layer1_wih_g)
    whh1 = _gate_major(layer1_whh_g)
    wih2 = _gate_major(layer2_wih_g)
    whh2 = _gate_major(layer2_whh_g)
    b1 = layer1_bias_g.reshape(1, G)
    b2 = layer2_bias_g.reshape(1, G)

    body = _make_body(T, BB, D, H, O, unroll=8)

    return pl.pallas_call(
        body,
        out_shape=jax.ShapeDtypeStruct((B, O), jnp.float32),
        grid=(NCORES,),
        in_specs=[
            pl.BlockSpec((T, BB, D), lambda i: (0, i, 0)),
            pl.BlockSpec((D, G), lambda i: (0, 0)),
            pl.BlockSpec((1, G), lambda i: (0, 0)),
            pl.BlockSpec((H, G), lambda i: (0, 0)),
            pl.BlockSpec((H, G), lambda i: (0, 0)),
            pl.BlockSpec((H, G), lambda i: (0, 0)),
            pl.BlockSpec((1, G), lambda i: (0, 0)),
            pl.BlockSpec((H, G), lambda i: (0, 0)),
            pl.BlockSpec((H, G), lambda i: (0, 0)),
            pl.BlockSpec((1, G), lambda i: (0, 0)),
            pl.BlockSpec((H, O), lambda i: (0, 0)),
            pl.BlockSpec((1, O), lambda i: (0, 0)),
        ],
        out_specs=pl.BlockSpec((BB, O), lambda i: (i, 0)),
        scratch_shapes=[
            pltpu.VMEM((T * BB, G), jnp.float32),
            pltpu.VMEM((T * BB, H), jnp.bfloat16),
        ],
        compiler_params=pltpu.CompilerParams(
            dimension_semantics=("parallel",)),
    )(x, layer0_w_ih_t, layer0_bias_flat, whh0,
      wih1, whh1, b1, wih2, whh2, b2, fc_w_t, fc_b)


# speedup vs baseline: 1.7071x; 1.6667x over previous
import jax
import jax.numpy as jnp
from jax import lax
from jax.experimental import pallas as pl
from jax.experimental.pallas import tpu as pltpu


def _make_body(T, B, D, H, O, unroll):
    G = 4 * H

    MC = 512

    def body(x_ref, w0_ref, b0_ref, whh0_ref, wih1_ref, whh1_ref, b1_ref,
             wih2_ref, whh2_ref, b2_ref, fcw_ref, fcb_ref, out_ref,
             xg_scr, h_all_scr):
        def proj(chunk_fn, w_ref, b_ref):
            for mc in range(0, T * B, MC):
                xg_scr[mc:mc + MC, :] = (
                    jnp.dot(chunk_fn(mc), w_ref[...],
                            preferred_element_type=jnp.float32)
                    + b_ref[...])

        TC = MC // B
        proj(lambda mc: x_ref[mc // B:mc // B + TC].reshape(MC, D),
             w0_ref, b0_ref)

        def run_layer(whh_ref, store_h):
            def step(t, carry):
                h, c = carry
                gates = (xg_scr[pl.ds(t * B, B), :]
                         + jnp.dot(h, whh_ref[...],
                                   preferred_element_type=jnp.float32))
                i = jax.nn.sigmoid(gates[:, 0:H])
                f = jax.nn.sigmoid(gates[:, H:2 * H])
                g = jnp.tanh(gates[:, 2 * H:3 * H])
                o = jax.nn.sigmoid(gates[:, 3 * H:4 * H])
                c_new = f * c + i * g
                h_new = (o * jnp.tanh(c_new)).astype(jnp.bfloat16)
                if store_h:
                    h_all_scr[pl.ds(t * B, B), :] = h_new
                return h_new, c_new

            z = (jnp.zeros((B, H), jnp.bfloat16),
                 jnp.zeros((B, H), jnp.float32))
            return lax.fori_loop(0, T, step, z, unroll=unroll)

        run_layer(whh0_ref, True)

        proj(lambda mc: h_all_scr[mc:mc + MC, :], wih1_ref, b1_ref)
        run_layer(whh1_ref, True)

        proj(lambda mc: h_all_scr[mc:mc + MC, :], wih2_ref, b2_ref)
        h_last, _ = run_layer(whh2_ref, False)

        out_ref[...] = (jnp.dot(h_last, fcw_ref[...],
                                preferred_element_type=jnp.float32)
                        + fcb_ref[...])

    return body


def _gate_major(w_g):
    g, hin, hout = w_g.shape
    return jnp.transpose(w_g, (1, 0, 2)).reshape(hin, g * hout)


def kernel(x_btd, layer0_w_ih_t, layer0_bias_flat, layer0_whh_g,
           layer1_wih_g, layer1_whh_g, layer1_bias_g,
           layer2_wih_g, layer2_whh_g, layer2_bias_g,
           fc_w_t, fc_b):
    B, T, D = x_btd.shape
    H = layer0_whh_g.shape[-1]
    O = fc_w_t.shape[1]
    G = 4 * H

    x = jnp.transpose(x_btd, (1, 0, 2)).astype(jnp.bfloat16)
    whh0 = _gate_major(layer0_whh_g)
    wih1 = _gate_major(layer1_wih_g)
    whh1 = _gate_major(layer1_whh_g)
    wih2 = _gate_major(layer2_wih_g)
    whh2 = _gate_major(layer2_whh_g)
    b1 = layer1_bias_g.reshape(1, G)
    b2 = layer2_bias_g.reshape(1, G)

    body = _make_body(T, B, D, H, O, unroll=8)

    return pl.pallas_call(
        body,
        out_shape=jax.ShapeDtypeStruct((B, O), jnp.float32),
        scratch_shapes=[
            pltpu.VMEM((T * B, G), jnp.float32),
            pltpu.VMEM((T * B, H), jnp.bfloat16),
        ],
    )(x, layer0_w_ih_t, layer0_bias_flat, whh0,
      wih1, whh1, b1, wih2, whh2, b2, fc_w_t, fc_b)
